# Initial kernel scaffold; baseline (speedup 1.0000x reference)
import functools

import jax
import jax.numpy as jnp
from jax import lax
from jax.experimental import pallas as pl
from jax.experimental.pallas import tpu as pltpu
from jax.experimental.pallas import tpu_sc as plsc

_N, _E, _D, _DE, _H = 10000, 320000, 128, 16, 128
_NC, _NS = 2, 16
_NW = _NC * _NS
_EW = _E // _NW
_CH = 128
_NFULL = _EW // _CH
_TAIL = _EW - _NFULL * _CH
_BLK_E = 4000
_NB_E = _E // _BLK_E
_BLK_N = 1000
_NB_N = _N // _BLK_N
_NPT = _N // _NS


def _prep_nodes_body(x_ref, w_ref, a_ref, b_ref):
    xb = x_ref[...]
    w = w_ref[...]
    a_ref[...] = jnp.dot(xb, w[:_D, :], preferred_element_type=jnp.float32)
    b_ref[...] = jnp.dot(xb, w[_D:2 * _D, :], preferred_element_type=jnp.float32)


def _prep_edges_body(ea_ref, w_ref, b1e_ref, c_ref):
    c_ref[...] = (
        jnp.dot(ea_ref[...], w_ref[2 * _D:, :], preferred_element_type=jnp.float32)
        + b1e_ref[...]
    )


def _fuse_w_body(w2e_ref, wm_ref, b2e_ref, bm_ref, w2m_ref, bm2_ref):
    w2m_ref[...] = jnp.dot(w2e_ref[...], wm_ref[...], preferred_element_type=jnp.float32)
    bm2_ref[...] = (
        jnp.dot(b2e_ref[...], wm_ref[...], preferred_element_type=jnp.float32)
        + bm_ref[...]
    )


def _edge_body(ag_ref, bg_ref, c_ref, w2m_ref, bm2_ref, g_ref, bb_ref,
               msg_ref, acc_ref):
    p = pl.program_id(0)
    i = pl.program_id(1)
    h = ag_ref[...] + bg_ref[...] + c_ref[...]

    @pl.when(jnp.logical_and(p == 0, i == 0))
    def _():
        acc_ref[...] = jnp.zeros_like(acc_ref)

    @pl.when(p == 0)
    def _():
        acc_ref[0, :] += jnp.sum(h, axis=0)
        acc_ref[1, :] += jnp.sum(h * h, axis=0)
        msg_ref[...] = jnp.zeros_like(msg_ref)

    @pl.when(p == 1)
    def _():
        mean = acc_ref[0, :] * (1.0 / _E)
        var = acc_ref[1, :] * (1.0 / _E) - mean * mean
        s = g_ref[0, :] * lax.rsqrt(var + 1e-5)
        t = bb_ref[0, :] - mean * s
        hn = jnp.maximum(h * s[None, :] + t[None, :], 0.0)
        m = jnp.dot(hn, w2m_ref[...], preferred_element_type=jnp.float32) + bm2_ref[...]
        msg_ref[...] = jnp.maximum(m, 0.0)


def _node_body(x_ref, a0_ref, a1_ref, w1n_ref, b1n_ref, g_ref, bb_ref,
               w2n_ref, b2n_ref, o_ref, acc_ref):
    p = pl.program_id(0)
    i = pl.program_id(1)
    agg = a0_ref[...] + a1_ref[...]
    w1n = w1n_ref[...]
    h2 = (
        jnp.dot(x_ref[...], w1n[:_D, :], preferred_element_type=jnp.float32)
        + jnp.dot(agg, w1n[_D:, :], preferred_element_type=jnp.float32)
        + b1n_ref[...]
    )

    @pl.when(jnp.logical_and(p == 0, i == 0))
    def _():
        acc_ref[...] = jnp.zeros_like(acc_ref)

    @pl.when(p == 0)
    def _():
        acc_ref[0, :] += jnp.sum(h2, axis=0)
        acc_ref[1, :] += jnp.sum(h2 * h2, axis=0)
        o_ref[...] = jnp.zeros_like(o_ref)

    @pl.when(p == 1)
    def _():
        mean = acc_ref[0, :] * (1.0 / _N)
        var = acc_ref[1, :] * (1.0 / _N) - mean * mean
        s = g_ref[0, :] * lax.rsqrt(var + 1e-5)
        t = bb_ref[0, :] - mean * s
        hn = jnp.maximum(h2 * s[None, :] + t[None, :], 0.0)
        o_ref[...] = (
            jnp.dot(hn, w2n_ref[...], preferred_element_type=jnp.float32)
            + b2n_ref[...]
        )


def _sc_gather(a_t, b_t, row, col):
    mesh = plsc.VectorSubcoreMesh(core_axis_name="c", subcore_axis_name="s")

    @functools.partial(
        pl.kernel,
        out_type=(
            jax.ShapeDtypeStruct((_E, _D), jnp.float32),
            jax.ShapeDtypeStruct((_E, _D), jnp.float32),
        ),
        mesh=mesh,
        scratch_types=[
            pltpu.VMEM((_CH,), jnp.int32),
            pltpu.VMEM((_CH,), jnp.int32),
            pltpu.VMEM((_CH, _D), jnp.float32),
            pltpu.VMEM((_CH, _D), jnp.float32),
            pltpu.VMEM((_TAIL,), jnp.int32),
            pltpu.VMEM((_TAIL,), jnp.int32),
            pltpu.VMEM((_TAIL, _D), jnp.float32),
            pltpu.VMEM((_TAIL, _D), jnp.float32),
            pltpu.SemaphoreType.DMA,
            pltpu.SemaphoreType.DMA,
        ],
    )
    def k(a_h, b_h, row_h, col_h, ag_h, bg_h,
          idxr, idxc, bufa, bufb, idxr2, idxc2, bufa2, bufb2, sema, semb):
        wid = lax.axis_index("s") * _NC + lax.axis_index("c")
        base = wid * _EW

        def chunk(off, idxr_, idxc_, bufa_, bufb_, sz):
            pltpu.sync_copy(row_h.at[pl.ds(off, sz)], idxr_)
            pltpu.sync_copy(col_h.at[pl.ds(off, sz)], idxc_)
            cpa = pltpu.async_copy(a_h.at[idxr_], bufa_, sema)
            cpb = pltpu.async_copy(b_h.at[idxc_], bufb_, semb)
            cpa.wait()
            cpb.wait()
            pltpu.sync_copy(bufa_, ag_h.at[pl.ds(off, sz)])
            pltpu.sync_copy(bufb_, bg_h.at[pl.ds(off, sz)])

        def body(j, carry):
            chunk(base + j * _CH, idxr, idxc, bufa, bufb, _CH)
            return carry

        lax.fori_loop(0, _NFULL, body, 0)
        chunk(base + _NFULL * _CH, idxr2, idxc2, bufa2, bufb2, _TAIL)

    return k(a_t, b_t, row, col)


def _sc_scatter(msg, row):
    mesh = plsc.VectorSubcoreMesh(core_axis_name="c", subcore_axis_name="s")

    @functools.partial(
        pl.kernel,
        out_type=jax.ShapeDtypeStruct((_NC * _N, _H), jnp.float32),
        mesh=mesh,
        scratch_types=[
            pltpu.VMEM((_CH, _H), jnp.float32),
            pltpu.VMEM((1, _CH), jnp.int32),
            pltpu.VMEM((_TAIL, _H), jnp.float32),
            pltpu.VMEM((1, _TAIL), jnp.int32),
            pltpu.VMEM((_CH, _H), jnp.float32),
            pltpu.VMEM_SHARED((_N, _H), jnp.float32),
        ],
    )
    def k(msg_h, row_h, agg_h, mbuf, ibuf, mbuf2, ibuf2, zbuf, agg_sh):
        c = lax.axis_index("c")
        s = lax.axis_index("s")
        wid = s * _NC + c

        def zb(i, carry):
            r = i // 8
            g = i % 8
            zbuf[r, pl.ds(g * 16, 16)] = jnp.zeros((16,), jnp.float32)
            return carry

        lax.fori_loop(0, _CH * 8, zb, 0)

        rows0 = s * _NPT
        spans = ((0, _CH), (_CH, _CH), (2 * _CH, _CH), (3 * _CH, _CH),
                 (4 * _CH, _NPT - 4 * _CH))
        for roff, sz in spans:
            pltpu.sync_copy(zbuf.at[pl.ds(0, sz)], agg_sh.at[pl.ds(rows0 + roff, sz)])
        plsc.subcore_barrier()

        base = wid * _EW

        def body(j, carry):
            off = base + j * _CH
            pltpu.sync_copy(row_h.at[pl.ds(off, _CH)], ibuf.at[0])
            pltpu.sync_copy(msg_h.at[pl.ds(off, _CH)], mbuf)
            pltpu.sync_copy(mbuf, agg_sh.at[ibuf.at[0]], add=True)
            return carry

        lax.fori_loop(0, _NFULL, body, 0)
        off = base + _NFULL * _CH
        pltpu.sync_copy(row_h.at[pl.ds(off, _TAIL)], ibuf2.at[0])
        pltpu.sync_copy(msg_h.at[pl.ds(off, _TAIL)], mbuf2)
        pltpu.sync_copy(mbuf2, agg_sh.at[ibuf2.at[0]], add=True)
        plsc.subcore_barrier()

        for roff, sz in spans:
            pltpu.sync_copy(agg_sh.at[pl.ds(rows0 + roff, sz)], zbuf.at[pl.ds(0, sz)])
            pltpu.sync_copy(zbuf.at[pl.ds(0, sz)],
                            agg_h.at[pl.ds(c * _N + rows0 + roff, sz)])

    return k(msg, row)


def kernel(x, edge_index, edge_attr, W1e, b1e, ge, be, W2e, b2e, Wm, bm,
           W1n, b1n, gn, bn, W2n, b2n):
    row = edge_index[0]
    col = edge_index[1]
    b1e2 = b1e.reshape(1, _H)
    ge2 = ge.reshape(1, _H)
    be2 = be.reshape(1, _H)
    b2e2 = b2e.reshape(1, _H)
    bm2d = bm.reshape(1, _H)
    b1n2 = b1n.reshape(1, _H)
    gn2 = gn.reshape(1, _H)
    bn2 = bn.reshape(1, _H)
    b2n2 = b2n.reshape(1, _H)

    a_t, b_t = pl.pallas_call(
        _prep_nodes_body,
        grid=(_NB_N,),
        in_specs=[
            pl.BlockSpec((_BLK_N, _D), lambda i: (i, 0)),
            pl.BlockSpec((2 * _D + _DE, _H), lambda i: (0, 0)),
        ],
        out_specs=[
            pl.BlockSpec((_BLK_N, _H), lambda i: (i, 0)),
            pl.BlockSpec((_BLK_N, _H), lambda i: (i, 0)),
        ],
        out_shape=[
            jax.ShapeDtypeStruct((_N, _H), jnp.float32),
            jax.ShapeDtypeStruct((_N, _H), jnp.float32),
        ],
    )(x, W1e)

    c_t = pl.pallas_call(
        _prep_edges_body,
        grid=(_NB_E,),
        in_specs=[
            pl.BlockSpec((_BLK_E, _DE), lambda i: (i, 0)),
            pl.BlockSpec((2 * _D + _DE, _H), lambda i: (0, 0)),
            pl.BlockSpec((1, _H), lambda i: (0, 0)),
        ],
        out_specs=pl.BlockSpec((_BLK_E, _H), lambda i: (i, 0)),
        out_shape=jax.ShapeDtypeStruct((_E, _H), jnp.float32),
    )(edge_attr, W1e, b1e2)

    w2m, bm2 = pl.pallas_call(
        _fuse_w_body,
        in_specs=[
            pl.BlockSpec((_H, _H), lambda: (0, 0)),
            pl.BlockSpec((_H, _H), lambda: (0, 0)),
            pl.BlockSpec((1, _H), lambda: (0, 0)),
            pl.BlockSpec((1, _H), lambda: (0, 0)),
        ],
        out_specs=[
            pl.BlockSpec((_H, _H), lambda: (0, 0)),
            pl.BlockSpec((1, _H), lambda: (0, 0)),
        ],
        out_shape=[
            jax.ShapeDtypeStruct((_H, _H), jnp.float32),
            jax.ShapeDtypeStruct((1, _H), jnp.float32),
        ],
    )(W2e, Wm, b2e2, bm2d)

    ag, bg = _sc_gather(a_t, b_t, row, col)

    msg = pl.pallas_call(
        _edge_body,
        grid=(2, _NB_E),
        in_specs=[
            pl.BlockSpec((_BLK_E, _H), lambda p, i: (i, 0)),
            pl.BlockSpec((_BLK_E, _H), lambda p, i: (i, 0)),
            pl.BlockSpec((_BLK_E, _H), lambda p, i: (i, 0)),
            pl.BlockSpec((_H, _H), lambda p, i: (0, 0)),
            pl.BlockSpec((1, _H), lambda p, i: (0, 0)),
            pl.BlockSpec((1, _H), lambda p, i: (0, 0)),
            pl.BlockSpec((1, _H), lambda p, i: (0, 0)),
        ],
        out_specs=pl.BlockSpec((_BLK_E, _H), lambda p, i: (i, 0)),
        out_shape=jax.ShapeDtypeStruct((_E, _H), jnp.float32),
        scratch_shapes=[pltpu.VMEM((8, _H), jnp.float32)],
    )(ag, bg, c_t, w2m, bm2, ge2, be2)

    aggp = _sc_scatter(msg, row)

    out = pl.pallas_call(
        _node_body,
        grid=(2, _NB_N),
        in_specs=[
            pl.BlockSpec((_BLK_N, _D), lambda p, i: (i, 0)),
            pl.BlockSpec((_BLK_N, _H), lambda p, i: (i, 0)),
            pl.BlockSpec((_BLK_N, _H), lambda p, i: (i + _NB_N, 0)),
            pl.BlockSpec((_D + _H, _H), lambda p, i: (0, 0)),
            pl.BlockSpec((1, _H), lambda p, i: (0, 0)),
            pl.BlockSpec((1, _H), lambda p, i: (0, 0)),
            pl.BlockSpec((1, _H), lambda p, i: (0, 0)),
            pl.BlockSpec((_H, _H), lambda p, i: (0, 0)),
            pl.BlockSpec((1, _H), lambda p, i: (0, 0)),
        ],
        out_specs=pl.BlockSpec((_BLK_N, _H), lambda p, i: (i, 0)),
        out_shape=jax.ShapeDtypeStruct((_N, _H), jnp.float32),
        scratch_shapes=[pltpu.VMEM((8, _H), jnp.float32)],
    )(x, aggp, aggp, W1n, b1n2, gn2, bn2, W2n, b2n2)

    return out


# speedup vs baseline: 2.6195x; 2.6195x over previous
import functools

import jax
import jax.numpy as jnp
from jax import lax
from jax.experimental import pallas as pl
from jax.experimental.pallas import tpu as pltpu
from jax.experimental.pallas import tpu_sc as plsc

_N, _E, _D, _DE, _H = 10000, 320000, 128, 16, 128
_NC, _NS = 2, 16
_NW = _NC * _NS
_EW = _E // _NW
_CH = 128
_NFULL = _EW // _CH
_TAIL = _EW - _NFULL * _CH
_BLK_E = 4000
_NB_E = _E // _BLK_E
_BLK_N = 1000
_NB_N = _N // _BLK_N
_NPT = 632
_NP = _NPT * _NS


def _prep_nodes_body(x_ref, w_ref, a_ref, b_ref):
    xb = x_ref[...]
    w = w_ref[...]
    a_ref[...] = jnp.dot(xb, w[:_D, :], preferred_element_type=jnp.float32)
    b_ref[...] = jnp.dot(xb, w[_D:2 * _D, :], preferred_element_type=jnp.float32)


def _prep_edges_body(ea_ref, w_ref, b1e_ref, c_ref):
    c_ref[...] = (
        jnp.dot(ea_ref[...], w_ref[2 * _D:, :], preferred_element_type=jnp.float32)
        + b1e_ref[...]
    )


def _fuse_w_body(w2e_ref, wm_ref, b2e_ref, bm_ref, w2m_ref, bm2_ref):
    w2m_ref[...] = jnp.dot(w2e_ref[...], wm_ref[...], preferred_element_type=jnp.float32)
    bm2_ref[...] = (
        jnp.dot(b2e_ref[...], wm_ref[...], preferred_element_type=jnp.float32)
        + bm_ref[...]
    )


def _edge_stats_body(ag_ref, bg_ref, c_ref, acc_ref):
    i = pl.program_id(0)
    h = ag_ref[...] + bg_ref[...] + c_ref[...]

    @pl.when(i == 0)
    def _():
        acc_ref[...] = jnp.zeros_like(acc_ref)

    acc_ref[0, :] += jnp.sum(h, axis=0)
    acc_ref[1, :] += jnp.sum(h * h, axis=0)


def _edge_apply_body(ag_ref, bg_ref, c_ref, acc_ref, w2m_ref, bm2_ref,
                     g_ref, bb_ref, msg_ref):
    h = ag_ref[...] + bg_ref[...] + c_ref[...]
    mean = acc_ref[0, :] * (1.0 / _E)
    var = acc_ref[1, :] * (1.0 / _E) - mean * mean
    s = g_ref[0, :] * lax.rsqrt(var + 1e-5)
    t = bb_ref[0, :] - mean * s
    hn = jnp.maximum(h * s[None, :] + t[None, :], 0.0)
    m = jnp.dot(hn, w2m_ref[...], preferred_element_type=jnp.float32) + bm2_ref[...]
    msg_ref[...] = jnp.maximum(m, 0.0)


def _h2(x_ref, a0_ref, a1_ref, w1n_ref, b1n_ref):
    agg = a0_ref[0] + a1_ref[0]
    w1n = w1n_ref[...]
    return (
        jnp.dot(x_ref[...], w1n[:_D, :], preferred_element_type=jnp.float32)
        + jnp.dot(agg, w1n[_D:, :], preferred_element_type=jnp.float32)
        + b1n_ref[...]
    )


def _node_stats_body(x_ref, a0_ref, a1_ref, w1n_ref, b1n_ref, acc_ref):
    i = pl.program_id(0)
    h2 = _h2(x_ref, a0_ref, a1_ref, w1n_ref, b1n_ref)

    @pl.when(i == 0)
    def _():
        acc_ref[...] = jnp.zeros_like(acc_ref)

    acc_ref[0, :] += jnp.sum(h2, axis=0)
    acc_ref[1, :] += jnp.sum(h2 * h2, axis=0)


def _node_apply_body(x_ref, a0_ref, a1_ref, w1n_ref, b1n_ref, acc_ref,
                     g_ref, bb_ref, w2n_ref, b2n_ref, o_ref):
    h2 = _h2(x_ref, a0_ref, a1_ref, w1n_ref, b1n_ref)
    mean = acc_ref[0, :] * (1.0 / _N)
    var = acc_ref[1, :] * (1.0 / _N) - mean * mean
    s = g_ref[0, :] * lax.rsqrt(var + 1e-5)
    t = bb_ref[0, :] - mean * s
    hn = jnp.maximum(h2 * s[None, :] + t[None, :], 0.0)
    o_ref[...] = (
        jnp.dot(hn, w2n_ref[...], preferred_element_type=jnp.float32)
        + b2n_ref[...]
    )


def _sc_gather(a_t, b_t, row, col):
    mesh = plsc.VectorSubcoreMesh(core_axis_name="c", subcore_axis_name="s")

    @functools.partial(
        pl.kernel,
        out_type=(
            jax.ShapeDtypeStruct((_E, _D), jnp.float32),
            jax.ShapeDtypeStruct((_E, _D), jnp.float32),
        ),
        mesh=mesh,
        scratch_types=[
            pltpu.VMEM((_CH,), jnp.int32),
            pltpu.VMEM((_CH,), jnp.int32),
            pltpu.VMEM((_CH, _D), jnp.float32),
            pltpu.VMEM((_CH, _D), jnp.float32),
            pltpu.VMEM((_TAIL,), jnp.int32),
            pltpu.VMEM((_TAIL,), jnp.int32),
            pltpu.VMEM((_TAIL, _D), jnp.float32),
            pltpu.VMEM((_TAIL, _D), jnp.float32),
            pltpu.SemaphoreType.DMA,
            pltpu.SemaphoreType.DMA,
        ],
    )
    def k(a_h, b_h, row_h, col_h, ag_h, bg_h,
          idxr, idxc, bufa, bufb, idxr2, idxc2, bufa2, bufb2, sema, semb):
        wid = lax.axis_index("s") * _NC + lax.axis_index("c")
        base = wid * _EW

        def chunk(off, idxr_, idxc_, bufa_, bufb_, sz):
            pltpu.sync_copy(row_h.at[pl.ds(off, sz)], idxr_)
            pltpu.sync_copy(col_h.at[pl.ds(off, sz)], idxc_)
            cpa = pltpu.async_copy(a_h.at[idxr_], bufa_, sema)
            cpb = pltpu.async_copy(b_h.at[idxc_], bufb_, semb)
            cpa.wait()
            cpb.wait()
            pltpu.sync_copy(bufa_, ag_h.at[pl.ds(off, sz)])
            pltpu.sync_copy(bufb_, bg_h.at[pl.ds(off, sz)])

        def body(j, carry):
            chunk(base + j * _CH, idxr, idxc, bufa, bufb, _CH)
            return carry

        lax.fori_loop(0, _NFULL, body, 0)
        chunk(base + _NFULL * _CH, idxr2, idxc2, bufa2, bufb2, _TAIL)

    return k(a_t, b_t, row, col)


def _sc_scatter(msg, row):
    mesh = plsc.VectorSubcoreMesh(core_axis_name="c", subcore_axis_name="s")

    @functools.partial(
        pl.kernel,
        out_type=jax.ShapeDtypeStruct((_NC, _NP, _H), jnp.float32),
        mesh=mesh,
        scratch_types=[
            pltpu.VMEM((_CH, _H), jnp.float32),
            pltpu.VMEM((1, _CH), jnp.int32),
            pltpu.VMEM((_TAIL, _H), jnp.float32),
            pltpu.VMEM((1, _TAIL), jnp.int32),
            pltpu.VMEM((_CH, _H), jnp.float32),
            pltpu.VMEM_SHARED((_NP, _H), jnp.float32),
        ],
    )
    def k(msg_h, row_h, agg_h, mbuf, ibuf, mbuf2, ibuf2, zbuf, agg_sh):
        c = lax.axis_index("c")
        s = lax.axis_index("s")
        wid = s * _NC + c

        def zb(i, carry):
            r = i // 8
            g = i % 8
            zbuf[r, pl.ds(g * 16, 16)] = jnp.zeros((16,), jnp.float32)
            return carry

        lax.fori_loop(0, _CH * 8, zb, 0)

        rows0 = s * _NPT
        spans = ((0, _CH), (_CH, _CH), (2 * _CH, _CH), (3 * _CH, _CH),
                 (4 * _CH, _NPT - 4 * _CH))
        for roff, sz in spans:
            pltpu.sync_copy(zbuf.at[pl.ds(0, sz)], agg_sh.at[pl.ds(rows0 + roff, sz)])
        plsc.subcore_barrier()

        base = wid * _EW

        def body(j, carry):
            off = base + j * _CH
            pltpu.sync_copy(row_h.at[pl.ds(off, _CH)], ibuf.at[0])
            pltpu.sync_copy(msg_h.at[pl.ds(off, _CH)], mbuf)
            pltpu.sync_copy(mbuf, agg_sh.at[ibuf.at[0]], add=True)
            return carry

        lax.fori_loop(0, _NFULL, body, 0)
        off = base + _NFULL * _CH
        pltpu.sync_copy(row_h.at[pl.ds(off, _TAIL)], ibuf2.at[0])
        pltpu.sync_copy(msg_h.at[pl.ds(off, _TAIL)], mbuf2)
        pltpu.sync_copy(mbuf2, agg_sh.at[ibuf2.at[0]], add=True)
        plsc.subcore_barrier()

        for roff, sz in spans:
            pltpu.sync_copy(agg_sh.at[pl.ds(rows0 + roff, sz)], zbuf.at[pl.ds(0, sz)])
            pltpu.sync_copy(zbuf.at[pl.ds(0, sz)],
                            agg_h.at[c, pl.ds(rows0 + roff, sz)])

    return k(msg, row)


def kernel(x, edge_index, edge_attr, W1e, b1e, ge, be, W2e, b2e, Wm, bm,
           W1n, b1n, gn, bn, W2n, b2n):
    row = edge_index[0]
    col = edge_index[1]
    b1e2 = b1e.reshape(1, _H)
    ge2 = ge.reshape(1, _H)
    be2 = be.reshape(1, _H)
    b2e2 = b2e.reshape(1, _H)
    bm2d = bm.reshape(1, _H)
    b1n2 = b1n.reshape(1, _H)
    gn2 = gn.reshape(1, _H)
    bn2 = bn.reshape(1, _H)
    b2n2 = b2n.reshape(1, _H)

    a_t, b_t = pl.pallas_call(
        _prep_nodes_body,
        grid=(_NB_N,),
        in_specs=[
            pl.BlockSpec((_BLK_N, _D), lambda i: (i, 0)),
            pl.BlockSpec((2 * _D + _DE, _H), lambda i: (0, 0)),
        ],
        out_specs=[
            pl.BlockSpec((_BLK_N, _H), lambda i: (i, 0)),
            pl.BlockSpec((_BLK_N, _H), lambda i: (i, 0)),
        ],
        out_shape=[
            jax.ShapeDtypeStruct((_N, _H), jnp.float32),
            jax.ShapeDtypeStruct((_N, _H), jnp.float32),
        ],
    )(x, W1e)

    c_t = pl.pallas_call(
        _prep_edges_body,
        grid=(_NB_E,),
        in_specs=[
            pl.BlockSpec((_BLK_E, _DE), lambda i: (i, 0)),
            pl.BlockSpec((2 * _D + _DE, _H), lambda i: (0, 0)),
            pl.BlockSpec((1, _H), lambda i: (0, 0)),
        ],
        out_specs=pl.BlockSpec((_BLK_E, _H), lambda i: (i, 0)),
        out_shape=jax.ShapeDtypeStruct((_E, _H), jnp.float32),
    )(edge_attr, W1e, b1e2)

    w2m, bm2 = pl.pallas_call(
        _fuse_w_body,
        in_specs=[
            pl.BlockSpec((_H, _H), lambda: (0, 0)),
            pl.BlockSpec((_H, _H), lambda: (0, 0)),
            pl.BlockSpec((1, _H), lambda: (0, 0)),
            pl.BlockSpec((1, _H), lambda: (0, 0)),
        ],
        out_specs=[
            pl.BlockSpec((_H, _H), lambda: (0, 0)),
            pl.BlockSpec((1, _H), lambda: (0, 0)),
        ],
        out_shape=[
            jax.ShapeDtypeStruct((_H, _H), jnp.float32),
            jax.ShapeDtypeStruct((1, _H), jnp.float32),
        ],
    )(W2e, Wm, b2e2, bm2d)

    ag, bg = _sc_gather(a_t, b_t, row, col)

    eblk = lambda i: (i, 0)
    full = lambda i: (0, 0)
    acc_e = pl.pallas_call(
        _edge_stats_body,
        grid=(_NB_E,),
        in_specs=[
            pl.BlockSpec((_BLK_E, _H), eblk),
            pl.BlockSpec((_BLK_E, _H), eblk),
            pl.BlockSpec((_BLK_E, _H), eblk),
        ],
        out_specs=pl.BlockSpec((8, _H), full),
        out_shape=jax.ShapeDtypeStruct((8, _H), jnp.float32),
    )(ag, bg, c_t)

    msg = pl.pallas_call(
        _edge_apply_body,
        grid=(_NB_E,),
        in_specs=[
            pl.BlockSpec((_BLK_E, _H), eblk),
            pl.BlockSpec((_BLK_E, _H), eblk),
            pl.BlockSpec((_BLK_E, _H), eblk),
            pl.BlockSpec((8, _H), full),
            pl.BlockSpec((_H, _H), full),
            pl.BlockSpec((1, _H), full),
            pl.BlockSpec((1, _H), full),
            pl.BlockSpec((1, _H), full),
        ],
        out_specs=pl.BlockSpec((_BLK_E, _H), eblk),
        out_shape=jax.ShapeDtypeStruct((_E, _H), jnp.float32),
    )(ag, bg, c_t, acc_e, w2m, bm2, ge2, be2)

    aggp = _sc_scatter(msg, row)

    nspecs = [
        pl.BlockSpec((_BLK_N, _D), eblk),
        pl.BlockSpec((1, _BLK_N, _H), lambda i: (0, i, 0)),
        pl.BlockSpec((1, _BLK_N, _H), lambda i: (1, i, 0)),
        pl.BlockSpec((_D + _H, _H), full),
        pl.BlockSpec((1, _H), full),
    ]
    acc_n = pl.pallas_call(
        _node_stats_body,
        grid=(_NB_N,),
        in_specs=nspecs,
        out_specs=pl.BlockSpec((8, _H), full),
        out_shape=jax.ShapeDtypeStruct((8, _H), jnp.float32),
    )(x, aggp, aggp, W1n, b1n2)

    out = pl.pallas_call(
        _node_apply_body,
        grid=(_NB_N,),
        in_specs=nspecs + [
            pl.BlockSpec((8, _H), full),
            pl.BlockSpec((1, _H), full),
            pl.BlockSpec((1, _H), full),
            pl.BlockSpec((_H, _H), full),
            pl.BlockSpec((1, _H), full),
        ],
        out_specs=pl.BlockSpec((_BLK_N, _H), eblk),
        out_shape=jax.ShapeDtypeStruct((_N, _H), jnp.float32),
    )(x, aggp, aggp, W1n, b1n2, acc_n, gn2, bn2, W2n, b2n2)

    return out


# speedup vs baseline: 2.8758x; 1.0978x over previous
import functools

import jax
import jax.numpy as jnp
from jax import lax
from jax.experimental import pallas as pl
from jax.experimental.pallas import tpu as pltpu
from jax.experimental.pallas import tpu_sc as plsc

_N, _E, _D, _DE, _H = 10000, 320000, 128, 16, 128
_NC, _NS = 2, 16
_NW = _NC * _NS
_EW = _E // _NW
_CH = 128
_NFULL = _EW // _CH
_TAIL = _EW - _NFULL * _CH
_BLK_E = 4000
_NB_E = _E // _BLK_E
_BLK_N = 1000
_NB_N = _N // _BLK_N
_NPT = 632
_NP = _NPT * _NS


def _prep_nodes_body(x_ref, w_ref, a_ref, b_ref):
    xb = x_ref[...]
    w = w_ref[...]
    a_ref[...] = jnp.dot(xb, w[:_D, :], preferred_element_type=jnp.float32)
    b_ref[...] = jnp.dot(xb, w[_D:2 * _D, :], preferred_element_type=jnp.float32)


def _prep_edges_body(ea_ref, w_ref, b1e_ref, c_ref):
    c_ref[...] = (
        jnp.dot(ea_ref[...], w_ref[2 * _D:, :], preferred_element_type=jnp.float32)
        + b1e_ref[...]
    )


def _fuse_w_body(w2e_ref, wm_ref, b2e_ref, bm_ref, w2m_ref, bm2_ref):
    w2m_ref[...] = jnp.dot(w2e_ref[...], wm_ref[...], preferred_element_type=jnp.float32)
    bm2_ref[...] = (
        jnp.dot(b2e_ref[...], wm_ref[...], preferred_element_type=jnp.float32)
        + bm_ref[...]
    )


def _edge_stats_body(ag_ref, bg_ref, c_ref, acc_ref):
    i = pl.program_id(0)
    h = ag_ref[...] + bg_ref[...] + c_ref[...]

    @pl.when(i == 0)
    def _():
        acc_ref[...] = jnp.zeros_like(acc_ref)

    acc_ref[0, :] += jnp.sum(h, axis=0)
    acc_ref[1, :] += jnp.sum(h * h, axis=0)


def _edge_apply_body(ag_ref, bg_ref, c_ref, acc_ref, w2m_ref, bm2_ref,
                     g_ref, bb_ref, msg_ref):
    h = ag_ref[...] + bg_ref[...] + c_ref[...]
    mean = acc_ref[0, :] * (1.0 / _E)
    var = acc_ref[1, :] * (1.0 / _E) - mean * mean
    s = g_ref[0, :] * lax.rsqrt(var + 1e-5)
    t = bb_ref[0, :] - mean * s
    hn = jnp.maximum(h * s[None, :] + t[None, :], 0.0)
    m = jnp.dot(hn, w2m_ref[...], preferred_element_type=jnp.float32) + bm2_ref[...]
    msg_ref[...] = jnp.maximum(m, 0.0)


def _h2(x_ref, a0_ref, a1_ref, w1n_ref, b1n_ref):
    agg = a0_ref[0] + a1_ref[0]
    w1n = w1n_ref[...]
    return (
        jnp.dot(x_ref[...], w1n[:_D, :], preferred_element_type=jnp.float32)
        + jnp.dot(agg, w1n[_D:, :], preferred_element_type=jnp.float32)
        + b1n_ref[...]
    )


def _node_stats_body(x_ref, a0_ref, a1_ref, w1n_ref, b1n_ref, acc_ref):
    i = pl.program_id(0)
    h2 = _h2(x_ref, a0_ref, a1_ref, w1n_ref, b1n_ref)

    @pl.when(i == 0)
    def _():
        acc_ref[...] = jnp.zeros_like(acc_ref)

    acc_ref[0, :] += jnp.sum(h2, axis=0)
    acc_ref[1, :] += jnp.sum(h2 * h2, axis=0)


def _node_apply_body(x_ref, a0_ref, a1_ref, w1n_ref, b1n_ref, acc_ref,
                     g_ref, bb_ref, w2n_ref, b2n_ref, o_ref):
    h2 = _h2(x_ref, a0_ref, a1_ref, w1n_ref, b1n_ref)
    mean = acc_ref[0, :] * (1.0 / _N)
    var = acc_ref[1, :] * (1.0 / _N) - mean * mean
    s = g_ref[0, :] * lax.rsqrt(var + 1e-5)
    t = bb_ref[0, :] - mean * s
    hn = jnp.maximum(h2 * s[None, :] + t[None, :], 0.0)
    o_ref[...] = (
        jnp.dot(hn, w2n_ref[...], preferred_element_type=jnp.float32)
        + b2n_ref[...]
    )


_GK = 400
_GNC = _EW // _GK
_GSUB = ((0, 128), (128, 128), (256, 128), (384, 16))
_SSUB = 2
_SK = _SSUB * _CH
_SNC = _EW // _SK
_ZR = 64


def _sc_gather(a_t, b_t, row, col):
    mesh = plsc.VectorSubcoreMesh(core_axis_name="c", subcore_axis_name="s")

    @functools.partial(
        pl.kernel,
        out_type=(
            jax.ShapeDtypeStruct((_E, _D), jnp.float32),
            jax.ShapeDtypeStruct((_E, _D), jnp.float32),
        ),
        mesh=mesh,
        scratch_types=[
            pltpu.VMEM((_GK,), jnp.int32),
            pltpu.VMEM((_GK,), jnp.int32),
            pltpu.VMEM((_GK, _D), jnp.float32),
            pltpu.VMEM((_GK, _D), jnp.float32),
            pltpu.SemaphoreType.DMA,
            pltpu.SemaphoreType.DMA,
            pltpu.SemaphoreType.DMA,
            pltpu.SemaphoreType.DMA,
        ],
    )
    def k(a_h, b_h, row_h, col_h, ag_h, bg_h,
          idxr, idxc, bufa, bufb, semi, sema, semb, semw):
        wid = lax.axis_index("s") * _NC + lax.axis_index("c")
        base = wid * _EW

        def body(j, carry):
            off = base + j * _GK
            ci1 = pltpu.async_copy(row_h.at[pl.ds(off, _GK)], idxr, semi)
            ci2 = pltpu.async_copy(col_h.at[pl.ds(off, _GK)], idxc, semi)
            ci1.wait()
            ci2.wait()
            cps = []
            for o, sz in _GSUB:
                cps.append(pltpu.async_copy(
                    a_h.at[idxr.at[pl.ds(o, sz)]], bufa.at[pl.ds(o, sz)], sema))
                cps.append(pltpu.async_copy(
                    b_h.at[idxc.at[pl.ds(o, sz)]], bufb.at[pl.ds(o, sz)], semb))
            for cp in cps:
                cp.wait()
            w1 = pltpu.async_copy(bufa, ag_h.at[pl.ds(off, _GK)], semw)
            w2 = pltpu.async_copy(bufb, bg_h.at[pl.ds(off, _GK)], semw)
            w1.wait()
            w2.wait()
            return carry

        lax.fori_loop(0, _GNC, body, 0)

    return k(a_t, b_t, row, col)


def _sc_scatter(msg, row):
    mesh = plsc.VectorSubcoreMesh(core_axis_name="c", subcore_axis_name="s")

    @functools.partial(
        pl.kernel,
        out_type=jax.ShapeDtypeStruct((_NC, _NP, _H), jnp.float32),
        mesh=mesh,
        scratch_types=[
            pltpu.VMEM((_SK, _H), jnp.float32),
            pltpu.VMEM((_SSUB, _CH), jnp.int32),
            pltpu.VMEM((_TAIL, _H), jnp.float32),
            pltpu.VMEM((1, _TAIL), jnp.int32),
            pltpu.VMEM((_ZR, _H), jnp.float32),
            pltpu.VMEM_SHARED((_NP, _H), jnp.float32),
            pltpu.SemaphoreType.DMA,
            pltpu.SemaphoreType.DMA,
        ],
    )
    def k(msg_h, row_h, agg_h, mbuf, ibuf, mbuf2, ibuf2, zbuf, agg_sh,
          semm, semi):
        c = lax.axis_index("c")
        s = lax.axis_index("s")
        wid = s * _NC + c

        def zb(i, carry):
            r = i // 8
            g = i % 8
            zbuf[r, pl.ds(g * 16, 16)] = jnp.zeros((16,), jnp.float32)
            return carry

        lax.fori_loop(0, _ZR * 8, zb, 0)

        rows0 = s * _NPT
        spans = tuple((i * _ZR, _ZR) for i in range(_NPT // _ZR)) + (
            ((_NPT // _ZR) * _ZR, _NPT % _ZR),)
        for roff, sz in spans:
            pltpu.sync_copy(zbuf.at[pl.ds(0, sz)], agg_sh.at[pl.ds(rows0 + roff, sz)])
        plsc.subcore_barrier()

        base = wid * _EW

        def body(j, carry):
            off = base + j * _SK
            cm = pltpu.async_copy(msg_h.at[pl.ds(off, _SK)], mbuf, semm)
            cis = [pltpu.async_copy(row_h.at[pl.ds(off + r * _CH, _CH)],
                                    ibuf.at[r], semi)
                   for r in range(_SSUB)]
            for ci in cis:
                ci.wait()
            cm.wait()
            for r in range(_SSUB):
                pltpu.sync_copy(mbuf.at[pl.ds(r * _CH, _CH)],
                                agg_sh.at[ibuf.at[r]], add=True)
            return carry

        lax.fori_loop(0, _SNC, body, 0)
        off = base + _SNC * _SK
        pltpu.sync_copy(row_h.at[pl.ds(off, _TAIL)], ibuf2.at[0])
        pltpu.sync_copy(msg_h.at[pl.ds(off, _TAIL)], mbuf2)
        pltpu.sync_copy(mbuf2, agg_sh.at[ibuf2.at[0]], add=True)
        plsc.subcore_barrier()

        for roff, sz in spans:
            pltpu.sync_copy(agg_sh.at[pl.ds(rows0 + roff, sz)], zbuf.at[pl.ds(0, sz)])
            pltpu.sync_copy(zbuf.at[pl.ds(0, sz)],
                            agg_h.at[c, pl.ds(rows0 + roff, sz)])

    return k(msg, row)


def kernel(x, edge_index, edge_attr, W1e, b1e, ge, be, W2e, b2e, Wm, bm,
           W1n, b1n, gn, bn, W2n, b2n):
    row = edge_index[0]
    col = edge_index[1]
    b1e2 = b1e.reshape(1, _H)
    ge2 = ge.reshape(1, _H)
    be2 = be.reshape(1, _H)
    b2e2 = b2e.reshape(1, _H)
    bm2d = bm.reshape(1, _H)
    b1n2 = b1n.reshape(1, _H)
    gn2 = gn.reshape(1, _H)
    bn2 = bn.reshape(1, _H)
    b2n2 = b2n.reshape(1, _H)

    a_t, b_t = pl.pallas_call(
        _prep_nodes_body,
        grid=(_NB_N,),
        in_specs=[
            pl.BlockSpec((_BLK_N, _D), lambda i: (i, 0)),
            pl.BlockSpec((2 * _D + _DE, _H), lambda i: (0, 0)),
        ],
        out_specs=[
            pl.BlockSpec((_BLK_N, _H), lambda i: (i, 0)),
            pl.BlockSpec((_BLK_N, _H), lambda i: (i, 0)),
        ],
        out_shape=[
            jax.ShapeDtypeStruct((_N, _H), jnp.float32),
            jax.ShapeDtypeStruct((_N, _H), jnp.float32),
        ],
    )(x, W1e)

    c_t = pl.pallas_call(
        _prep_edges_body,
        grid=(_NB_E,),
        in_specs=[
            pl.BlockSpec((_BLK_E, _DE), lambda i: (i, 0)),
            pl.BlockSpec((2 * _D + _DE, _H), lambda i: (0, 0)),
            pl.BlockSpec((1, _H), lambda i: (0, 0)),
        ],
        out_specs=pl.BlockSpec((_BLK_E, _H), lambda i: (i, 0)),
        out_shape=jax.ShapeDtypeStruct((_E, _H), jnp.float32),
    )(edge_attr, W1e, b1e2)

    w2m, bm2 = pl.pallas_call(
        _fuse_w_body,
        in_specs=[
            pl.BlockSpec((_H, _H), lambda: (0, 0)),
            pl.BlockSpec((_H, _H), lambda: (0, 0)),
            pl.BlockSpec((1, _H), lambda: (0, 0)),
            pl.BlockSpec((1, _H), lambda: (0, 0)),
        ],
        out_specs=[
            pl.BlockSpec((_H, _H), lambda: (0, 0)),
            pl.BlockSpec((1, _H), lambda: (0, 0)),
        ],
        out_shape=[
            jax.ShapeDtypeStruct((_H, _H), jnp.float32),
            jax.ShapeDtypeStruct((1, _H), jnp.float32),
        ],
    )(W2e, Wm, b2e2, bm2d)

    ag, bg = _sc_gather(a_t, b_t, row, col)

    eblk = lambda i: (i, 0)
    full = lambda i: (0, 0)
    acc_e = pl.pallas_call(
        _edge_stats_body,
        grid=(_NB_E,),
        in_specs=[
            pl.BlockSpec((_BLK_E, _H), eblk),
            pl.BlockSpec((_BLK_E, _H), eblk),
            pl.BlockSpec((_BLK_E, _H), eblk),
        ],
        out_specs=pl.BlockSpec((8, _H), full),
        out_shape=jax.ShapeDtypeStruct((8, _H), jnp.float32),
    )(ag, bg, c_t)

    msg = pl.pallas_call(
        _edge_apply_body,
        grid=(_NB_E,),
        in_specs=[
            pl.BlockSpec((_BLK_E, _H), eblk),
            pl.BlockSpec((_BLK_E, _H), eblk),
            pl.BlockSpec((_BLK_E, _H), eblk),
            pl.BlockSpec((8, _H), full),
            pl.BlockSpec((_H, _H), full),
            pl.BlockSpec((1, _H), full),
            pl.BlockSpec((1, _H), full),
            pl.BlockSpec((1, _H), full),
        ],
        out_specs=pl.BlockSpec((_BLK_E, _H), eblk),
        out_shape=jax.ShapeDtypeStruct((_E, _H), jnp.float32),
    )(ag, bg, c_t, acc_e, w2m, bm2, ge2, be2)

    aggp = _sc_scatter(msg, row)

    nspecs = [
        pl.BlockSpec((_BLK_N, _D), eblk),
        pl.BlockSpec((1, _BLK_N, _H), lambda i: (0, i, 0)),
        pl.BlockSpec((1, _BLK_N, _H), lambda i: (1, i, 0)),
        pl.BlockSpec((_D + _H, _H), full),
        pl.BlockSpec((1, _H), full),
    ]
    acc_n = pl.pallas_call(
        _node_stats_body,
        grid=(_NB_N,),
        in_specs=nspecs,
        out_specs=pl.BlockSpec((8, _H), full),
        out_shape=jax.ShapeDtypeStruct((8, _H), jnp.float32),
    )(x, aggp, aggp, W1n, b1n2)

    out = pl.pallas_call(
        _node_apply_body,
        grid=(_NB_N,),
        in_specs=nspecs + [
            pl.BlockSpec((8, _H), full),
            pl.BlockSpec((1, _H), full),
            pl.BlockSpec((1, _H), full),
            pl.BlockSpec((_H, _H), full),
            pl.BlockSpec((1, _H), full),
        ],
        out_specs=pl.BlockSpec((_BLK_N, _H), eblk),
        out_shape=jax.ShapeDtypeStruct((_N, _H), jnp.float32),
    )(x, aggp, aggp, W1n, b1n2, acc_n, gn2, bn2, W2n, b2n2)

    return out


# speedup vs baseline: 4.2372x; 1.4734x over previous
import functools

import jax
import jax.numpy as jnp
from jax import lax
from jax.experimental import pallas as pl
from jax.experimental.pallas import tpu as pltpu
from jax.experimental.pallas import tpu_sc as plsc

_N, _E, _D, _DE, _H = 10000, 320000, 128, 16, 128
_NC, _NS = 2, 16
_NW = _NC * _NS
_EW = _E // _NW
_CH = 128
_NFULL = _EW // _CH
_TAIL = _EW - _NFULL * _CH
_BLK_E = 4000
_NB_E = _E // _BLK_E
_BLK_N = 1000
_NB_N = _N // _BLK_N
_BLK_P = 2000
_NB_P = _N // _BLK_P
_NPT = 632
_NP = _NPT * _NS


def _prep_nodes_body(x_ref, w_ref, a_ref, b_ref):
    xb = x_ref[...]
    w = w_ref[...]
    a_ref[...] = jnp.dot(xb, w[:_D, :], preferred_element_type=jnp.float32)
    b_ref[...] = jnp.dot(xb, w[_D:2 * _D, :], preferred_element_type=jnp.float32)


def _fuse_w_body(w2e_ref, wm_ref, b2e_ref, bm_ref, w2m_ref, bm2_ref):
    w2m_ref[...] = jnp.dot(w2e_ref[...], wm_ref[...], preferred_element_type=jnp.float32)
    bm2_ref[...] = (
        jnp.dot(b2e_ref[...], wm_ref[...], preferred_element_type=jnp.float32)
        + bm_ref[...]
    )


def _edge_h(ag_ref, bg_ref, ea_ref, w_ref, b1e_ref):
    cb = (
        jnp.dot(ea_ref[...], w_ref[2 * _D:, :], preferred_element_type=jnp.float32)
        + b1e_ref[...]
    )
    return ag_ref[...] + bg_ref[...] + cb


def _edge_stats_body(ag_ref, bg_ref, ea_ref, w_ref, b1e_ref, h_ref, acc_ref):
    i = pl.program_id(0)
    h = _edge_h(ag_ref, bg_ref, ea_ref, w_ref, b1e_ref)
    h_ref[...] = h.astype(jnp.bfloat16)

    @pl.when(i == 0)
    def _():
        acc_ref[...] = jnp.zeros_like(acc_ref)

    acc_ref[0, :] += jnp.sum(h, axis=0)
    acc_ref[1, :] += jnp.sum(h * h, axis=0)


def _edge_apply_body(h_ref, acc_ref, w2m_ref, bm2_ref, g_ref, bb_ref, msg_ref):
    h = h_ref[...].astype(jnp.float32)
    mean = acc_ref[0, :] * (1.0 / _E)
    var = acc_ref[1, :] * (1.0 / _E) - mean * mean
    s = g_ref[0, :] * lax.rsqrt(var + 1e-5)
    t = bb_ref[0, :] - mean * s
    hn = jnp.maximum(h * s[None, :] + t[None, :], 0.0)
    m = jnp.dot(hn, w2m_ref[...], preferred_element_type=jnp.float32) + bm2_ref[...]
    msg_ref[...] = jnp.maximum(m, 0.0)


def _h2(x_ref, a0_ref, a1_ref, w1n_ref, b1n_ref):
    agg = a0_ref[0] + a1_ref[0]
    w1n = w1n_ref[...]
    return (
        jnp.dot(x_ref[...], w1n[:_D, :], preferred_element_type=jnp.float32)
        + jnp.dot(agg, w1n[_D:, :], preferred_element_type=jnp.float32)
        + b1n_ref[...]
    )


def _node_stats_body(x_ref, a0_ref, a1_ref, w1n_ref, b1n_ref, acc_ref):
    i = pl.program_id(0)
    h2 = _h2(x_ref, a0_ref, a1_ref, w1n_ref, b1n_ref)

    @pl.when(i == 0)
    def _():
        acc_ref[...] = jnp.zeros_like(acc_ref)

    acc_ref[0, :] += jnp.sum(h2, axis=0)
    acc_ref[1, :] += jnp.sum(h2 * h2, axis=0)


def _node_apply_body(x_ref, a0_ref, a1_ref, w1n_ref, b1n_ref, acc_ref,
                     g_ref, bb_ref, w2n_ref, b2n_ref, o_ref):
    h2 = _h2(x_ref, a0_ref, a1_ref, w1n_ref, b1n_ref)
    mean = acc_ref[0, :] * (1.0 / _N)
    var = acc_ref[1, :] * (1.0 / _N) - mean * mean
    s = g_ref[0, :] * lax.rsqrt(var + 1e-5)
    t = bb_ref[0, :] - mean * s
    hn = jnp.maximum(h2 * s[None, :] + t[None, :], 0.0)
    o_ref[...] = (
        jnp.dot(hn, w2n_ref[...], preferred_element_type=jnp.float32)
        + b2n_ref[...]
    )


_GK = 200
_GNC = _EW // _GK
_GSUB = ((0, 128), (128, 72))
_SNC = _EW // _CH
_ZR = 64


def _sc_gather(a_t, b_t, row, col):
    mesh = plsc.VectorSubcoreMesh(core_axis_name="c", subcore_axis_name="s")

    @functools.partial(
        pl.kernel,
        out_type=(
            jax.ShapeDtypeStruct((_E, _D), jnp.float32),
            jax.ShapeDtypeStruct((_E, _D), jnp.float32),
        ),
        mesh=mesh,
        scratch_types=[
            pltpu.VMEM((_GK,), jnp.int32),
            pltpu.VMEM((_GK,), jnp.int32),
            pltpu.VMEM((_GK,), jnp.int32),
            pltpu.VMEM((_GK,), jnp.int32),
            pltpu.VMEM((_GK, _D), jnp.float32),
            pltpu.VMEM((_GK, _D), jnp.float32),
            pltpu.VMEM((_GK, _D), jnp.float32),
            pltpu.VMEM((_GK, _D), jnp.float32),
            pltpu.SemaphoreType.DMA,
            pltpu.SemaphoreType.DMA,
            pltpu.SemaphoreType.DMA,
            pltpu.SemaphoreType.DMA,
            pltpu.SemaphoreType.DMA,
        ],
    )
    def k(a_h, b_h, row_h, col_h, ag_h, bg_h,
          idxr0, idxc0, idxr1, idxc1, bufa0, bufb0, bufa1, bufb1,
          semi, semg0, semg1, semw0, semw1):
        wid = lax.axis_index("s") * _NC + lax.axis_index("c")
        base = wid * _EW
        sets = ((idxr0, idxc0, bufa0, bufb0, semg0, semw0),
                (idxr1, idxc1, bufa1, bufb1, semg1, semw1))

        for b in (0, 1):
            _, _, bufa_, bufb_, _, semw_ = sets[b]
            poff = base + b * _GK
            pltpu.async_copy(bufa_, ag_h.at[pl.ds(poff, _GK)], semw_)
            pltpu.async_copy(bufb_, bg_h.at[pl.ds(poff, _GK)], semw_)

        def body(t, carry):
            for b in (0, 1):
                idxr_, idxc_, bufa_, bufb_, semg_, semw_ = sets[b]
                off = base + (2 * t + b) * _GK
                pltpu.make_async_copy(bufa_, ag_h.at[pl.ds(off, _GK)], semw_).wait()
                pltpu.make_async_copy(bufb_, bg_h.at[pl.ds(off, _GK)], semw_).wait()
                ci1 = pltpu.async_copy(row_h.at[pl.ds(off, _GK)], idxr_, semi)
                ci2 = pltpu.async_copy(col_h.at[pl.ds(off, _GK)], idxc_, semi)
                ci1.wait()
                ci2.wait()
                cps = []
                for o, sz in _GSUB:
                    cps.append(pltpu.async_copy(
                        a_h.at[idxr_.at[pl.ds(o, sz)]], bufa_.at[pl.ds(o, sz)],
                        semg_))
                    cps.append(pltpu.async_copy(
                        b_h.at[idxc_.at[pl.ds(o, sz)]], bufb_.at[pl.ds(o, sz)],
                        semg_))
                for cp in cps:
                    cp.wait()
                pltpu.async_copy(bufa_, ag_h.at[pl.ds(off, _GK)], semw_)
                pltpu.async_copy(bufb_, bg_h.at[pl.ds(off, _GK)], semw_)
            return carry

        lax.fori_loop(0, _GNC // 2, body, 0)
        for b in (0, 1):
            _, _, bufa_, bufb_, _, semw_ = sets[b]
            poff = base + b * _GK
            pltpu.make_async_copy(bufa_, ag_h.at[pl.ds(poff, _GK)], semw_).wait()
            pltpu.make_async_copy(bufb_, bg_h.at[pl.ds(poff, _GK)], semw_).wait()

    return k(a_t, b_t, row, col)


def _sc_scatter(msg, row):
    mesh = plsc.VectorSubcoreMesh(core_axis_name="c", subcore_axis_name="s")

    @functools.partial(
        pl.kernel,
        out_type=jax.ShapeDtypeStruct((_NC, _NP, _H), jnp.float32),
        mesh=mesh,
        scratch_types=[
            pltpu.VMEM((_CH, _H), jnp.float32),
            pltpu.VMEM((_CH, _H), jnp.float32),
            pltpu.VMEM((1, _CH), jnp.int32),
            pltpu.VMEM((1, _CH), jnp.int32),
            pltpu.VMEM((_TAIL, _H), jnp.float32),
            pltpu.VMEM((1, _TAIL), jnp.int32),
            pltpu.VMEM((_ZR, _H), jnp.float32),
            pltpu.VMEM_SHARED((_NP, _H), jnp.float32),
            pltpu.SemaphoreType.DMA,
            pltpu.SemaphoreType.DMA,
            pltpu.SemaphoreType.DMA,
            pltpu.SemaphoreType.DMA,
        ],
    )
    def k(msg_h, row_h, agg_h, mbuf0, mbuf1, ibuf0, ibuf1, mbuf2, ibuf2,
          zbuf, agg_sh, semm0, semm1, semi0, semi1):
        c = lax.axis_index("c")
        s = lax.axis_index("s")
        wid = s * _NC + c

        def zb(i, carry):
            r = i // 8
            g = i % 8
            zbuf[r, pl.ds(g * 16, 16)] = jnp.zeros((16,), jnp.float32)
            return carry

        lax.fori_loop(0, _ZR * 8, zb, 0)

        rows0 = s * _NPT
        spans = tuple((i * _ZR, _ZR) for i in range(_NPT // _ZR)) + (
            ((_NPT // _ZR) * _ZR, _NPT % _ZR),)
        for roff, sz in spans:
            pltpu.sync_copy(zbuf.at[pl.ds(0, sz)], agg_sh.at[pl.ds(rows0 + roff, sz)])
        plsc.subcore_barrier()

        base = wid * _EW
        sets = ((ibuf0, mbuf0, semi0, semm0), (ibuf1, mbuf1, semi1, semm1))

        pltpu.async_copy(row_h.at[pl.ds(base, _CH)], ibuf0.at[0], semi0)
        pltpu.async_copy(msg_h.at[pl.ds(base, _CH)], mbuf0, semm0)

        def body(t, carry):
            for b in (0, 1):
                ibuf_, mbuf_, semi_, semm_ = sets[b]
                nibuf, nmbuf, nsemi, nsemm = sets[1 - b]
                j = 2 * t + b
                off = base + j * _CH
                pltpu.make_async_copy(
                    row_h.at[pl.ds(off, _CH)], ibuf_.at[0], semi_).wait()
                pltpu.make_async_copy(
                    msg_h.at[pl.ds(off, _CH)], mbuf_, semm_).wait()
                noff = base + jnp.minimum(j + 1, _SNC - 1) * _CH
                pltpu.async_copy(row_h.at[pl.ds(noff, _CH)], nibuf.at[0], nsemi)
                pltpu.async_copy(msg_h.at[pl.ds(noff, _CH)], nmbuf, nsemm)
                pltpu.sync_copy(mbuf_, agg_sh.at[ibuf_.at[0]], add=True)
            return carry

        lax.fori_loop(0, _SNC // 2, body, 0)
        pltpu.make_async_copy(row_h.at[pl.ds(base, _CH)], ibuf0.at[0], semi0).wait()
        pltpu.make_async_copy(msg_h.at[pl.ds(base, _CH)], mbuf0, semm0).wait()
        off = base + _SNC * _CH
        pltpu.sync_copy(row_h.at[pl.ds(off, _TAIL)], ibuf2.at[0])
        pltpu.sync_copy(msg_h.at[pl.ds(off, _TAIL)], mbuf2)
        pltpu.sync_copy(mbuf2, agg_sh.at[ibuf2.at[0]], add=True)
        plsc.subcore_barrier()

        for roff, sz in spans:
            pltpu.sync_copy(agg_sh.at[pl.ds(rows0 + roff, sz)], zbuf.at[pl.ds(0, sz)])
            pltpu.sync_copy(zbuf.at[pl.ds(0, sz)],
                            agg_h.at[c, pl.ds(rows0 + roff, sz)])

    return k(msg, row)


def kernel(x, edge_index, edge_attr, W1e, b1e, ge, be, W2e, b2e, Wm, bm,
           W1n, b1n, gn, bn, W2n, b2n):
    row = edge_index[0]
    col = edge_index[1]
    b1e2 = b1e.reshape(1, _H)
    ge2 = ge.reshape(1, _H)
    be2 = be.reshape(1, _H)
    b2e2 = b2e.reshape(1, _H)
    bm2d = bm.reshape(1, _H)
    b1n2 = b1n.reshape(1, _H)
    gn2 = gn.reshape(1, _H)
    bn2 = bn.reshape(1, _H)
    b2n2 = b2n.reshape(1, _H)

    a_t, b_t = pl.pallas_call(
        _prep_nodes_body,
        grid=(_NB_P,),
        in_specs=[
            pl.BlockSpec((_BLK_P, _D), lambda i: (i, 0)),
            pl.BlockSpec((2 * _D + _DE, _H), lambda i: (0, 0)),
        ],
        out_specs=[
            pl.BlockSpec((_BLK_P, _H), lambda i: (i, 0)),
            pl.BlockSpec((_BLK_P, _H), lambda i: (i, 0)),
        ],
        out_shape=[
            jax.ShapeDtypeStruct((_N, _H), jnp.float32),
            jax.ShapeDtypeStruct((_N, _H), jnp.float32),
        ],
    )(x, W1e)

    w2m, bm2 = pl.pallas_call(
        _fuse_w_body,
        in_specs=[
            pl.BlockSpec((_H, _H), lambda: (0, 0)),
            pl.BlockSpec((_H, _H), lambda: (0, 0)),
            pl.BlockSpec((1, _H), lambda: (0, 0)),
            pl.BlockSpec((1, _H), lambda: (0, 0)),
        ],
        out_specs=[
            pl.BlockSpec((_H, _H), lambda: (0, 0)),
            pl.BlockSpec((1, _H), lambda: (0, 0)),
        ],
        out_shape=[
            jax.ShapeDtypeStruct((_H, _H), jnp.float32),
            jax.ShapeDtypeStruct((1, _H), jnp.float32),
        ],
    )(W2e, Wm, b2e2, bm2d)

    ag, bg = _sc_gather(a_t, b_t, row, col)

    eblk = lambda i: (i, 0)
    full = lambda i: (0, 0)
    especs = [
        pl.BlockSpec((_BLK_E, _H), eblk),
        pl.BlockSpec((_BLK_E, _H), eblk),
        pl.BlockSpec((_BLK_E, _DE), eblk),
        pl.BlockSpec((2 * _D + _DE, _H), full),
        pl.BlockSpec((1, _H), full),
    ]
    h_bf, acc_e = pl.pallas_call(
        _edge_stats_body,
        grid=(_NB_E,),
        in_specs=especs,
        out_specs=[
            pl.BlockSpec((_BLK_E, _H), eblk),
            pl.BlockSpec((8, _H), full),
        ],
        out_shape=[
            jax.ShapeDtypeStruct((_E, _H), jnp.bfloat16),
            jax.ShapeDtypeStruct((8, _H), jnp.float32),
        ],
    )(ag, bg, edge_attr, W1e, b1e2)

    msg = pl.pallas_call(
        _edge_apply_body,
        grid=(_NB_E,),
        in_specs=[
            pl.BlockSpec((_BLK_E, _H), eblk),
            pl.BlockSpec((8, _H), full),
            pl.BlockSpec((_H, _H), full),
            pl.BlockSpec((1, _H), full),
            pl.BlockSpec((1, _H), full),
            pl.BlockSpec((1, _H), full),
        ],
        out_specs=pl.BlockSpec((_BLK_E, _H), eblk),
        out_shape=jax.ShapeDtypeStruct((_E, _H), jnp.float32),
    )(h_bf, acc_e, w2m, bm2, ge2, be2)

    aggp = _sc_scatter(msg, row)

    nspecs = [
        pl.BlockSpec((_BLK_N, _D), eblk),
        pl.BlockSpec((1, _BLK_N, _H), lambda i: (0, i, 0)),
        pl.BlockSpec((1, _BLK_N, _H), lambda i: (1, i, 0)),
        pl.BlockSpec((_D + _H, _H), full),
        pl.BlockSpec((1, _H), full),
    ]
    acc_n = pl.pallas_call(
        _node_stats_body,
        grid=(_NB_N,),
        in_specs=nspecs,
        out_specs=pl.BlockSpec((8, _H), full),
        out_shape=jax.ShapeDtypeStruct((8, _H), jnp.float32),
    )(x, aggp, aggp, W1n, b1n2)

    out = pl.pallas_call(
        _node_apply_body,
        grid=(_NB_N,),
        in_specs=nspecs + [
            pl.BlockSpec((8, _H), full),
            pl.BlockSpec((1, _H), full),
            pl.BlockSpec((1, _H), full),
            pl.BlockSpec((_H, _H), full),
            pl.BlockSpec((1, _H), full),
        ],
        out_specs=pl.BlockSpec((_BLK_N, _H), eblk),
        out_shape=jax.ShapeDtypeStruct((_N, _H), jnp.float32),
    )(x, aggp, aggp, W1n, b1n2, acc_n, gn2, bn2, W2n, b2n2)

    return out
